# baseline (speedup 1.0000x reference)
import functools

import jax
import jax.numpy as jnp
from jax.experimental import pallas as pl
from jax.experimental.pallas import tpu as pltpu
from jax.experimental.pallas import tpu_sc as plsc

_EPS = 1e-6
_GATHER_WINDOW = 128


def _sc_gather(table, idx2d):
    n_idx = idx2d.shape[1]
    w = table.shape[1]
    mesh = plsc.VectorSubcoreMesh(core_axis_name="c", subcore_axis_name="s")

    @functools.partial(
        pl.kernel,
        out_type=jax.ShapeDtypeStruct((n_idx, w), table.dtype),
        mesh=mesh,
    )
    def gather_kernel(table_hbm, i_hbm, o_hbm):
        def body(i_vmem, o_vmem):
            pltpu.sync_copy(table_hbm.at[i_vmem.at[0]], o_vmem)

        pltpu.emit_pipeline(
            body,
            grid=(n_idx // _GATHER_WINDOW,),
            in_specs=[
                pl.BlockSpec((1, _GATHER_WINDOW), index_map=lambda i: (0, i))
            ],
            out_specs=[
                pl.BlockSpec((_GATHER_WINDOW, w), index_map=lambda i: (i, 0))
            ],
            core_axis_name=("c", "s"),
            dimension_semantics=(pltpu.PARALLEL,),
        )(i_hbm, o_hbm)

    return gather_kernel(table, idx2d)


def _pad_body(t_ref, o_ref):
    rows, hidden = t_ref.shape
    o_ref[...] = jnp.concatenate(
        [t_ref[...], jnp.zeros((rows, 128 - hidden), t_ref.dtype)], axis=1
    )


def _mask_body(x_ref, m_ref):
    bb, l = x_ref.shape
    m = x_ref[...] > 0
    m_ref[...] = jnp.broadcast_to(m[:, None, None, :], (bb, 1, l, l))


def _ln_body(tok_ref, pos_ref, g_ref, b_ref, o_ref):
    hidden = o_ref.shape[-1]
    h = tok_ref[..., :hidden] + pos_ref[...][None]
    mean = jnp.mean(h, axis=-1, keepdims=True)
    c = h - mean
    var = jnp.sum(c * c, axis=-1, keepdims=True) / (hidden - 1)
    std = jnp.sqrt(var)
    o_ref[...] = g_ref[...][None, None] * (c / (std + _EPS)) + b_ref[...][None, None]


def kernel(x, token_table, pos_table, gamma, beta):
    b, l = x.shape
    _, hidden = token_table.shape

    vocab = token_table.shape[0]
    rblk = 2000
    table128 = pl.pallas_call(
        _pad_body,
        grid=(vocab // rblk,),
        in_specs=[pl.BlockSpec((rblk, hidden), lambda i: (i, 0))],
        out_specs=pl.BlockSpec((rblk, 128), lambda i: (i, 0)),
        out_shape=jax.ShapeDtypeStruct((vocab, 128), token_table.dtype),
    )(token_table)
    idx2d = x.reshape(1, b * l).astype(jnp.int32)
    tok = _sc_gather(table128, idx2d).reshape(b, l, 128)

    bb = 8
    mask = pl.pallas_call(
        _mask_body,
        grid=(b // bb,),
        in_specs=[pl.BlockSpec((bb, l), lambda i: (i, 0))],
        out_specs=pl.BlockSpec((bb, 1, l, l), lambda i: (i, 0, 0, 0)),
        out_shape=jax.ShapeDtypeStruct((b, 1, l, l), jnp.bool_),
    )(x)

    out = pl.pallas_call(
        _ln_body,
        grid=(b // bb,),
        in_specs=[
            pl.BlockSpec((bb, l, 128), lambda i: (i, 0, 0)),
            pl.BlockSpec((l, hidden), lambda i: (0, 0)),
            pl.BlockSpec((hidden,), lambda i: (0,)),
            pl.BlockSpec((hidden,), lambda i: (0,)),
        ],
        out_specs=pl.BlockSpec((bb, l, hidden), lambda i: (i, 0, 0)),
        out_shape=jax.ShapeDtypeStruct((b, l, hidden), jnp.float32),
    )(tok, pos_table, gamma, beta)

    return (out, mask)


# speedup vs baseline: 1.5723x; 1.5723x over previous
import functools

import jax
import jax.numpy as jnp
from jax.experimental import pallas as pl
from jax.experimental.pallas import tpu as pltpu
from jax.experimental.pallas import tpu_sc as plsc

_EPS = 1e-6
_GATHER_WINDOW = 128


def _sc_gather(table, idx2d):
    n_idx = idx2d.shape[1]
    w = table.shape[1]
    mesh = plsc.VectorSubcoreMesh(core_axis_name="c", subcore_axis_name="s")

    @functools.partial(
        pl.kernel,
        out_type=jax.ShapeDtypeStruct((n_idx, w), table.dtype),
        mesh=mesh,
    )
    def gather_kernel(table_hbm, i_hbm, o_hbm):
        def body(i_vmem, o_vmem):
            pltpu.sync_copy(table_hbm.at[i_vmem.at[0]], o_vmem)

        pltpu.emit_pipeline(
            body,
            grid=(n_idx // _GATHER_WINDOW,),
            in_specs=[
                pl.BlockSpec((1, _GATHER_WINDOW), index_map=lambda i: (0, i))
            ],
            out_specs=[
                pl.BlockSpec((_GATHER_WINDOW, w), index_map=lambda i: (i, 0))
            ],
            core_axis_name=("c", "s"),
            dimension_semantics=(pltpu.PARALLEL,),
        )(i_hbm, o_hbm)

    return gather_kernel(table, idx2d)


def _pad_body(t_ref, o_ref):
    rows, hidden = t_ref.shape
    o_ref[...] = jnp.concatenate(
        [t_ref[...], jnp.zeros((rows, 128 - hidden), t_ref.dtype)], axis=1
    )


def _mask_body(xt_ref, m_ref):
    l, b = xt_ref.shape
    ib = m_ref.shape[1]
    m = (xt_ref[...] > 0).astype(jnp.int8)
    m_ref[...] = jnp.broadcast_to(m[None, None, :, :], (1, ib, l, b))


def _ln_body(tok_ref, pos_ref, g_ref, b_ref, o_ref):
    hidden = o_ref.shape[-1]
    h = tok_ref[..., :hidden] + pos_ref[...][None]
    mean = jnp.mean(h, axis=-1, keepdims=True)
    c = h - mean
    var = jnp.sum(c * c, axis=-1, keepdims=True) / (hidden - 1)
    std = jnp.sqrt(var)
    o_ref[...] = g_ref[...][None, None] * (c / (std + _EPS)) + b_ref[...][None, None]


def kernel(x, token_table, pos_table, gamma, beta):
    b, l = x.shape
    _, hidden = token_table.shape

    vocab = token_table.shape[0]
    rblk = 2000
    table128 = pl.pallas_call(
        _pad_body,
        grid=(vocab // rblk,),
        in_specs=[pl.BlockSpec((rblk, hidden), lambda i: (i, 0))],
        out_specs=pl.BlockSpec((rblk, 128), lambda i: (i, 0)),
        out_shape=jax.ShapeDtypeStruct((vocab, 128), token_table.dtype),
    )(token_table)
    idx2d = x.reshape(1, b * l).astype(jnp.int32)
    tok = _sc_gather(table128, idx2d).reshape(b, l, 128)

    xt = jnp.transpose(x)
    ib = 25
    mask_t = pl.pallas_call(
        _mask_body,
        grid=(l // ib,),
        in_specs=[pl.BlockSpec((l, b), lambda i: (0, 0))],
        out_specs=pl.BlockSpec((1, ib, l, b), lambda i: (0, i, 0, 0)),
        out_shape=jax.ShapeDtypeStruct((1, l, l, b), jnp.int8),
    )(xt)
    mask = jnp.transpose(mask_t, (3, 0, 1, 2)).astype(jnp.bool_)

    bb = 8
    out = pl.pallas_call(
        _ln_body,
        grid=(b // bb,),
        in_specs=[
            pl.BlockSpec((bb, l, 128), lambda i: (i, 0, 0)),
            pl.BlockSpec((l, hidden), lambda i: (0, 0)),
            pl.BlockSpec((hidden,), lambda i: (0,)),
            pl.BlockSpec((hidden,), lambda i: (0,)),
        ],
        out_specs=pl.BlockSpec((bb, l, hidden), lambda i: (i, 0, 0)),
        out_shape=jax.ShapeDtypeStruct((b, l, hidden), jnp.float32),
    )(tok, pos_table, gamma, beta)

    return (out, mask)


# speedup vs baseline: 2.1542x; 1.3701x over previous
import functools

import jax
import jax.numpy as jnp
from jax.experimental import pallas as pl
from jax.experimental.pallas import tpu as pltpu
from jax.experimental.pallas import tpu_sc as plsc

_EPS = 1e-6
_GATHER_WINDOW = 128


def _sc_gather(table, idx2d):
    n_idx = idx2d.shape[1]
    w = table.shape[1]
    mesh = plsc.VectorSubcoreMesh(core_axis_name="c", subcore_axis_name="s")

    @functools.partial(
        pl.kernel,
        out_type=jax.ShapeDtypeStruct((n_idx, w), table.dtype),
        mesh=mesh,
    )
    def gather_kernel(table_hbm, i_hbm, o_hbm):
        def body(i_vmem, o_vmem):
            pltpu.sync_copy(table_hbm.at[i_vmem.at[0]], o_vmem)

        pltpu.emit_pipeline(
            body,
            grid=(n_idx // _GATHER_WINDOW,),
            in_specs=[
                pl.BlockSpec((1, _GATHER_WINDOW), index_map=lambda i: (0, i))
            ],
            out_specs=[
                pl.BlockSpec((_GATHER_WINDOW, w), index_map=lambda i: (i, 0))
            ],
            core_axis_name=("c", "s"),
            dimension_semantics=(pltpu.PARALLEL,),
        )(i_hbm, o_hbm)

    return gather_kernel(table, idx2d)


def _pad_body(t_ref, o_ref):
    rows, hidden = t_ref.shape
    o_ref[...] = jnp.concatenate(
        [t_ref[...], jnp.zeros((rows, 128 - hidden), t_ref.dtype)], axis=1
    )


def _mask_body(xt_ref, m_ref):
    l, b = xt_ref.shape
    ib = m_ref.shape[1]
    m = (xt_ref[...] > 0).astype(jnp.int8)
    m_ref[...] = jnp.broadcast_to(m[None, None, :, :], (1, ib, l, b))


def _ln_body(tok_ref, pos_ref, g_ref, b_ref, o_ref):
    lb, hidden, bc = o_ref.shape
    for j in range(lb):
        h_tok = tok_ref[:, j, :hidden] + pos_ref[j, :][None]
        h = jnp.transpose(h_tok)
        mean = jnp.mean(h, axis=0, keepdims=True)
        c = h - mean
        var = jnp.sum(c * c, axis=0, keepdims=True) / (hidden - 1)
        std = jnp.sqrt(var)
        o_ref[j] = g_ref[...] * (c / (std + _EPS)) + b_ref[...]


def kernel(x, token_table, pos_table, gamma, beta):
    b, l = x.shape
    _, hidden = token_table.shape

    vocab = token_table.shape[0]
    rblk = 2000
    table128 = pl.pallas_call(
        _pad_body,
        grid=(vocab // rblk,),
        in_specs=[pl.BlockSpec((rblk, hidden), lambda i: (i, 0))],
        out_specs=pl.BlockSpec((rblk, 128), lambda i: (i, 0)),
        out_shape=jax.ShapeDtypeStruct((vocab, 128), token_table.dtype),
    )(token_table)
    idx2d = x.reshape(1, b * l).astype(jnp.int32)
    tok = _sc_gather(table128, idx2d).reshape(b, l, 128)

    xt = jnp.transpose(x)
    ib = 25
    mask_t = pl.pallas_call(
        _mask_body,
        grid=(l // ib,),
        in_specs=[pl.BlockSpec((l, b), lambda i: (0, 0))],
        out_specs=pl.BlockSpec((1, ib, l, b), lambda i: (0, i, 0, 0)),
        out_shape=jax.ShapeDtypeStruct((1, l, l, b), jnp.int8),
    )(xt)
    mask = jnp.transpose(mask_t, (3, 0, 1, 2)).astype(jnp.bool_)

    lb = 8
    out_t = pl.pallas_call(
        _ln_body,
        grid=(l // lb,),
        in_specs=[
            pl.BlockSpec((b, lb, 128), lambda i: (0, i, 0)),
            pl.BlockSpec((lb, hidden), lambda i: (i, 0)),
            pl.BlockSpec((hidden, 1), lambda i: (0, 0)),
            pl.BlockSpec((hidden, 1), lambda i: (0, 0)),
        ],
        out_specs=pl.BlockSpec((lb, hidden, b), lambda i: (i, 0, 0)),
        out_shape=jax.ShapeDtypeStruct((l, hidden, b), jnp.float32),
    )(tok, pos_table, gamma.reshape(hidden, 1), beta.reshape(hidden, 1))
    out = jnp.transpose(out_t, (2, 0, 1))

    return (out, mask)


# speedup vs baseline: 2.4587x; 1.1413x over previous
import functools

import jax
import jax.numpy as jnp
from jax.experimental import pallas as pl
from jax.experimental.pallas import tpu as pltpu
from jax.experimental.pallas import tpu_sc as plsc

_EPS = 1e-6
_GATHER_WINDOW = 128


def _sc_gather(table, idx2d):
    n_idx = idx2d.shape[1]
    w = table.shape[1]
    mesh = plsc.VectorSubcoreMesh(core_axis_name="c", subcore_axis_name="s")

    @functools.partial(
        pl.kernel,
        out_type=jax.ShapeDtypeStruct((n_idx, w), table.dtype),
        mesh=mesh,
    )
    def gather_kernel(table_hbm, i_hbm, o_hbm):
        def body(i_vmem, o_vmem):
            pltpu.sync_copy(table_hbm.at[i_vmem.at[0]], o_vmem)

        pltpu.emit_pipeline(
            body,
            grid=(n_idx // _GATHER_WINDOW,),
            in_specs=[
                pl.BlockSpec((1, _GATHER_WINDOW), index_map=lambda i: (0, i))
            ],
            out_specs=[
                pl.BlockSpec((_GATHER_WINDOW, w), index_map=lambda i: (i, 0))
            ],
            core_axis_name=("c", "s"),
            dimension_semantics=(pltpu.PARALLEL,),
        )(i_hbm, o_hbm)

    return gather_kernel(table, idx2d)


def _pad_body(t_ref, o_ref):
    rows, hidden = t_ref.shape
    o_ref[...] = jnp.concatenate(
        [t_ref[...], jnp.zeros((rows, 128 - hidden), t_ref.dtype)], axis=1
    )


def _mask_body(xt_ref, m_ref):
    l, b = xt_ref.shape
    ib = m_ref.shape[1]
    m = (xt_ref[...] > 0).astype(jnp.int8)
    m_ref[...] = jnp.broadcast_to(m[None, None, :, :], (1, ib, l, b))


def _ln_body(tok_ref, pos_ref, g_ref, b_ref, o_ref):
    lg, hidden, bc = o_ref.shape
    ones_row = jnp.ones((1, hidden), jnp.float32)
    for j in range(lg):
        h_tok = tok_ref[j] + pos_ref[j]
        h = jnp.transpose(h_tok)[:hidden]
        s = jnp.dot(ones_row, h, preferred_element_type=jnp.float32)
        sq = jnp.dot(ones_row, h * h, preferred_element_type=jnp.float32)
        mean = s / hidden
        var = jnp.maximum(sq - s * mean, 0.0) / (hidden - 1)
        std = jnp.sqrt(var)
        o_ref[j] = g_ref[...] * ((h - mean) / (std + _EPS)) + b_ref[...]


def kernel(x, token_table, pos_table, gamma, beta):
    b, l = x.shape
    _, hidden = token_table.shape

    vocab = token_table.shape[0]
    rblk = 2000
    table128 = pl.pallas_call(
        _pad_body,
        grid=(vocab // rblk,),
        in_specs=[pl.BlockSpec((rblk, hidden), lambda i: (i, 0))],
        out_specs=pl.BlockSpec((rblk, 128), lambda i: (i, 0)),
        out_shape=jax.ShapeDtypeStruct((vocab, 128), token_table.dtype),
    )(token_table)
    xt = jnp.transpose(x)
    idx2d = xt.reshape(1, b * l).astype(jnp.int32)
    tok = _sc_gather(table128, idx2d).reshape(l, b, 128)

    ib = 25
    mask_t = pl.pallas_call(
        _mask_body,
        grid=(l // ib,),
        in_specs=[pl.BlockSpec((l, b), lambda i: (0, 0))],
        out_specs=pl.BlockSpec((1, ib, l, b), lambda i: (0, i, 0, 0)),
        out_shape=jax.ShapeDtypeStruct((1, l, l, b), jnp.int8),
    )(xt)
    mask = jnp.transpose(mask_t, (3, 0, 1, 2)).astype(jnp.bool_)

    lg = 8
    out_t = pl.pallas_call(
        _ln_body,
        grid=(l // lg,),
        in_specs=[
            pl.BlockSpec((lg, b, 128), lambda i: (i, 0, 0)),
            pl.BlockSpec((lg, 1, 128), lambda i: (i, 0, 0)),
            pl.BlockSpec((hidden, 1), lambda i: (0, 0)),
            pl.BlockSpec((hidden, 1), lambda i: (0, 0)),
        ],
        out_specs=pl.BlockSpec((lg, hidden, b), lambda i: (i, 0, 0)),
        out_shape=jax.ShapeDtypeStruct((l, hidden, b), jnp.float32),
    )(
        tok,
        jnp.pad(pos_table, ((0, 0), (0, 128 - hidden))).reshape(l, 1, 128),
        gamma.reshape(hidden, 1),
        beta.reshape(hidden, 1),
    )
    out = jnp.transpose(out_t, (2, 0, 1))

    return (out, mask)
